# baseline (speedup 1.0000x reference)
import functools

import jax
import jax.numpy as jnp
from jax import lax
from jax.experimental import pallas as pl
from jax.experimental.pallas import tpu as pltpu
from jax.experimental.pallas import tpu_sc as plsc

NUM_CORES = 2
NUM_SUBCORES = 16
NUM_WORKERS = NUM_CORES * NUM_SUBCORES

CHUNK = 256
NBUF = 4
DEPTH = 2


def _make_gather(batch, dim):
    b_per_w = batch // NUM_WORKERS
    n_chunks = b_per_w // CHUNK
    n_groups = n_chunks // NBUF
    mesh = plsc.VectorSubcoreMesh(core_axis_name="c", subcore_axis_name="s")

    @functools.partial(
        pl.kernel,
        out_type=jax.ShapeDtypeStruct((batch, dim), jnp.float32),
        mesh=mesh,
        scratch_types=[
            pltpu.VMEM((n_chunks, CHUNK), jnp.int32),
            pltpu.VMEM((NBUF, CHUNK, dim), jnp.float32),
            pltpu.SemaphoreType.DMA((NBUF,)),
            pltpu.SemaphoreType.DMA((NBUF,)),
        ],
        compiler_params=pltpu.CompilerParams(use_tc_tiling_on_sc=False),
    )
    def gather_kernel(idx_hbm, table_hbm, out_hbm, idx_v, bufs, sem_g, sem_w):
        wid = lax.axis_index("s") * NUM_CORES + lax.axis_index("c")
        row0 = wid * n_chunks
        base = wid * b_per_w

        pltpu.sync_copy(idx_hbm.at[pl.ds(row0, n_chunks)], idx_v)

        def gather_copy(c, buf):
            return pltpu.make_async_copy(
                table_hbm.at[idx_v.at[c]], bufs.at[buf], sem_g.at[buf])

        def write_copy(c, buf):
            return pltpu.make_async_copy(
                bufs.at[buf], out_hbm.at[pl.ds(base + c * CHUNK, CHUNK)],
                sem_w.at[buf])

        for b in range(DEPTH):
            gather_copy(b, b).start()

        def group(gg, carry):
            for b in range(NBUF):
                j = gg * NBUF + b
                gather_copy(j, b).wait()
                write_copy(j, b).start()
                r = j + DEPTH
                rbuf = (b + DEPTH) % NBUF

                @pl.when(jnp.logical_and(r >= NBUF, r < n_chunks))
                def _wait_prev():
                    write_copy(r - NBUF, rbuf).wait()

                @pl.when(r < n_chunks)
                def _refire():
                    gather_copy(r, rbuf).start()
            return carry

        lax.fori_loop(0, n_groups, group, 0)

        for b in range(NBUF):
            write_copy(n_chunks - NBUF + b, b).wait()

    return gather_kernel


def kernel(items, table):
    batch, hist = items.shape
    _, dim = table.shape
    total = batch * hist
    idx = items.reshape(total // CHUNK, CHUNK).astype(jnp.int32)
    gathered = _make_gather(total, dim)(idx, table)
    return gathered.reshape(batch, hist, dim)


# speedup vs baseline: 1.6238x; 1.6238x over previous
import functools

import jax
import jax.numpy as jnp
from jax import lax
from jax.experimental import pallas as pl
from jax.experimental.pallas import tpu as pltpu
from jax.experimental.pallas import tpu_sc as plsc

NUM_CORES = 2
NUM_SUBCORES = 16
NUM_WORKERS = NUM_CORES * NUM_SUBCORES

CB = 8
NBUF = 4
DEPTH = 2


def _make_gather(batch, hist, dim):
    b_per_w = batch // NUM_WORKERS
    n_groups = b_per_w // CB
    n_iters = n_groups // NBUF
    mesh = plsc.VectorSubcoreMesh(core_axis_name="c", subcore_axis_name="s")

    @functools.partial(
        pl.kernel,
        out_type=jax.ShapeDtypeStruct((batch, hist, dim), jnp.float32),
        mesh=mesh,
        scratch_types=[
            pltpu.VMEM((b_per_w, hist), jnp.int32),
            pltpu.VMEM((NBUF, CB, hist, dim), jnp.float32),
            pltpu.SemaphoreType.DMA((NBUF,)),
            pltpu.SemaphoreType.DMA((NBUF,)),
        ],
        compiler_params=pltpu.CompilerParams(use_tc_tiling_on_sc=False),
    )
    def gather_kernel(items_hbm, table_hbm, out_hbm, idx_v, bufs, sem_g, sem_w):
        wid = lax.axis_index("s") * NUM_CORES + lax.axis_index("c")
        base = wid * b_per_w

        pltpu.sync_copy(items_hbm.at[pl.ds(base, b_per_w)], idx_v)

        def gather_copy(g, kk, buf):
            return pltpu.make_async_copy(
                table_hbm.at[idx_v.at[g * CB + kk]], bufs.at[buf, kk],
                sem_g.at[buf])

        def write_copy(g, buf):
            return pltpu.make_async_copy(
                bufs.at[buf], out_hbm.at[pl.ds(base + g * CB, CB)],
                sem_w.at[buf])

        for g in range(DEPTH):
            for kk in range(CB):
                gather_copy(g, kk, g).start()

        def body(gg, carry):
            for b in range(NBUF):
                j = gg * NBUF + b
                for kk in range(CB):
                    gather_copy(j, kk, b).wait()
                write_copy(j, b).start()
                r = j + DEPTH
                rbuf = (b + DEPTH) % NBUF

                @pl.when(jnp.logical_and(r >= NBUF, r < n_groups))
                def _wait_prev():
                    write_copy(r - NBUF, rbuf).wait()

                @pl.when(r < n_groups)
                def _refire():
                    for kk in range(CB):
                        gather_copy(r, kk, rbuf).start()
            return carry

        lax.fori_loop(0, n_iters, body, 0)

        for b in range(NBUF):
            write_copy(n_groups - NBUF + b, b).wait()

    return gather_kernel


def kernel(items, table):
    batch, hist = items.shape
    _, dim = table.shape
    return _make_gather(batch, hist, dim)(items.astype(jnp.int32), table)


# speedup vs baseline: 2.2921x; 1.4115x over previous
import functools

import jax
import jax.numpy as jnp
from jax import lax
from jax.experimental import pallas as pl
from jax.experimental.pallas import tpu as pltpu
from jax.experimental.pallas import tpu_sc as plsc

NUM_CORES = 2
NUM_SUBCORES = 16
NUM_WORKERS = NUM_CORES * NUM_SUBCORES

CB = 8
NBUF = 4
DEPTH = 2


PAD_HIST = 56


def _make_gather(batch, hist, dim):
    b_per_w = batch // NUM_WORKERS
    n_groups = b_per_w // CB
    n_iters = n_groups // NBUF
    mesh = plsc.VectorSubcoreMesh(core_axis_name="c", subcore_axis_name="s")

    @functools.partial(
        pl.kernel,
        out_type=jax.ShapeDtypeStruct((batch, PAD_HIST, 128), jnp.float32),
        mesh=mesh,
        scratch_types=[
            pltpu.VMEM((b_per_w, hist), jnp.int32),
            pltpu.VMEM((NBUF, CB, hist, dim), jnp.float32),
            pltpu.SemaphoreType.DMA((NBUF,)),
            pltpu.SemaphoreType.DMA((NBUF,)),
        ],
        compiler_params=pltpu.CompilerParams(use_tc_tiling_on_sc=False),
    )
    def gather_kernel(items_hbm, table_hbm, out_hbm, idx_v, bufs, sem_g, sem_w):
        wid = lax.axis_index("s") * NUM_CORES + lax.axis_index("c")
        base = wid * b_per_w

        pltpu.sync_copy(items_hbm.at[pl.ds(base, b_per_w)], idx_v)

        def gather_copy(g, kk, buf):
            return pltpu.make_async_copy(
                table_hbm.at[idx_v.at[g * CB + kk]], bufs.at[buf, kk],
                sem_g.at[buf])

        def write_copy(g, buf):
            return pltpu.make_async_copy(
                bufs.at[buf],
                out_hbm.at[pl.ds(base + g * CB, CB), pl.ds(0, hist),
                           pl.ds(0, dim)],
                sem_w.at[buf])

        for g in range(DEPTH):
            for kk in range(CB):
                gather_copy(g, kk, g).start()

        def body(gg, carry):
            for b in range(NBUF):
                j = gg * NBUF + b
                for kk in range(CB):
                    gather_copy(j, kk, b).wait()
                write_copy(j, b).start()
                r = j + DEPTH
                rbuf = (b + DEPTH) % NBUF

                @pl.when(jnp.logical_and(r >= NBUF, r < n_groups))
                def _wait_prev():
                    write_copy(r - NBUF, rbuf).wait()

                @pl.when(r < n_groups)
                def _refire():
                    for kk in range(CB):
                        gather_copy(r, kk, rbuf).start()
            return carry

        lax.fori_loop(0, n_iters, body, 0)

        for b in range(NBUF):
            write_copy(n_groups - NBUF + b, b).wait()

    return gather_kernel


def kernel(items, table):
    batch, hist = items.shape
    _, dim = table.shape
    padded = _make_gather(batch, hist, dim)(items.astype(jnp.int32), table)
    return padded[:, :hist, :dim]
